# baseline (speedup 1.0000x reference)
import jax
import jax.numpy as jnp
from jax import lax
from jax.experimental import pallas as pl
from jax.experimental.pallas import tpu as pltpu
from jax.experimental.pallas import tpu_sc as plsc

TOP_K = 2
N_TOKENS = 8192
HIDDEN = 2048
N_SLOTS = N_TOKENS * TOP_K

NUM_WORKERS = 32
TOK_PER_WORKER = N_TOKENS // NUM_WORKERS
SLOT_PER_WORKER = TOP_K * TOK_PER_WORKER
CHUNK_T = 4
CHUNK_R = CHUNK_T * TOP_K
N_CHUNKS = TOK_PER_WORKER // CHUNK_T
N_PAIRS = N_CHUNKS // 2
LANES = 16
D_VECS = HIDDEN // LANES


def _sc_body(table_hbm, idx_hbm, sb_hbm, out_hbm, idx_v, sb_v, rows_v,
             out_v, gsem0, gsem1, wsem0, wsem1):
    c = lax.axis_index("c")
    s = lax.axis_index("s")
    wid = s * 2 + c
    slot_base = wid * SLOT_PER_WORKER
    tok_base = wid * TOK_PER_WORKER

    pltpu.sync_copy(idx_hbm.at[pl.ds(slot_base, SLOT_PER_WORKER)], idx_v)
    pltpu.sync_copy(sb_hbm.at[pl.ds(slot_base, SLOT_PER_WORKER)], sb_v)

    gsems = (gsem0, gsem1)
    wsems = (wsem0, wsem1)

    def start_gather(g, b):
        idx_ref = idx_v.at[pl.ds(g * CHUNK_R, CHUNK_R)]
        pltpu.async_copy(table_hbm.at[idx_ref], rows_v.at[b], gsems[b])

    def wait_gather(g, b):
        idx_ref = idx_v.at[pl.ds(g * CHUNK_R, CHUNK_R)]
        pltpu.make_async_copy(table_hbm.at[idx_ref], rows_v.at[b],
                              gsems[b]).wait()

    def compute(g, b):
        scales = [sb_v[g * CHUNK_R + r] for r in range(CHUNK_R)]

        def d_body(d, dcarry):
            col = pl.ds(d * LANES, LANES)
            for t in range(CHUNK_T):
                r0 = rows_v[b, 2 * t, col]
                r1 = rows_v[b, 2 * t + 1, col]
                out_v[b, t, col] = (scales[2 * t] * r0
                                    + scales[2 * t + 1] * r1)
            return dcarry

        lax.fori_loop(0, D_VECS, d_body, 0)

    def start_write(g, b):
        pltpu.async_copy(out_v.at[b],
                         out_hbm.at[pl.ds(tok_base + g * CHUNK_T, CHUNK_T)],
                         wsems[b])

    def wait_write(b):
        pltpu.make_async_copy(out_v.at[b],
                              out_hbm.at[pl.ds(tok_base, CHUNK_T)],
                              wsems[b]).wait()

    start_gather(0, 0)

    def pair_body(j, carry):
        g0 = 2 * j
        start_gather(g0 + 1, 1)
        wait_gather(g0, 0)

        @pl.when(j > 0)
        def _():
            wait_write(0)

        compute(g0, 0)

        @pl.when(j < N_PAIRS - 1)
        def _():
            start_gather(g0 + 2, 0)

        start_write(g0, 0)
        wait_gather(g0 + 1, 1)

        @pl.when(j > 0)
        def _():
            wait_write(1)

        compute(g0 + 1, 1)
        start_write(g0 + 1, 1)
        return carry

    lax.fori_loop(0, N_PAIRS, pair_body, 0)
    wait_write(0)
    wait_write(1)


@jax.jit
def kernel(moe_output, scores, mapped_slots):
    idx = mapped_slots.astype(jnp.int32)
    scores_b = jnp.broadcast_to(scores[:, None], (N_SLOTS, LANES))

    mesh = plsc.VectorSubcoreMesh(core_axis_name="c", subcore_axis_name="s",
                                  num_cores=2, num_subcores=16)
    run = pl.kernel(
        _sc_body,
        out_type=jax.ShapeDtypeStruct((N_TOKENS, HIDDEN), jnp.float32),
        mesh=mesh,
        scratch_types=[
            pltpu.VMEM((SLOT_PER_WORKER,), jnp.int32),
            pltpu.VMEM((SLOT_PER_WORKER, LANES), jnp.float32),
            pltpu.VMEM((2, CHUNK_R, HIDDEN), jnp.float32),
            pltpu.VMEM((2, CHUNK_T, HIDDEN), jnp.float32),
            pltpu.SemaphoreType.DMA,
            pltpu.SemaphoreType.DMA,
            pltpu.SemaphoreType.DMA,
            pltpu.SemaphoreType.DMA,
        ],
    )
    return run(moe_output, idx, scores_b)


# speedup vs baseline: 2.2820x; 2.2820x over previous
import jax
import jax.numpy as jnp
from jax import lax
from jax.experimental import pallas as pl
from jax.experimental.pallas import tpu as pltpu
from jax.experimental.pallas import tpu_sc as plsc

TOP_K = 2
N_TOKENS = 8192
HIDDEN = 2048
N_SLOTS = N_TOKENS * TOP_K

NUM_WORKERS = 32
TOK_PER_WORKER = N_TOKENS // NUM_WORKERS
SLOT_PER_WORKER = TOP_K * TOK_PER_WORKER
CHUNK_T = 4
CHUNK_R = CHUNK_T * TOP_K
N_CHUNKS = TOK_PER_WORKER // CHUNK_T
N_PAIRS = N_CHUNKS // 2
LANES = 16
D_VECS = HIDDEN // LANES


def _sc_body(table_hbm, idx_hbm, sb_hbm, out_hbm, idx_v, sb_v, rows_v,
             out_v, gsem0, gsem1, wsem0, wsem1):
    c = lax.axis_index("c")
    s = lax.axis_index("s")
    wid = s * 2 + c
    slot_base = wid * SLOT_PER_WORKER
    tok_base = wid * TOK_PER_WORKER

    pltpu.sync_copy(idx_hbm.at[pl.ds(slot_base, SLOT_PER_WORKER)], idx_v)
    pltpu.sync_copy(sb_hbm.at[pl.ds(slot_base, SLOT_PER_WORKER)], sb_v)

    gsems = (gsem0, gsem1)
    wsems = (wsem0, wsem1)

    def start_gather(g, b):
        idx_ref = idx_v.at[pl.ds(g * CHUNK_R, CHUNK_R)]
        pltpu.async_copy(table_hbm.at[idx_ref], rows_v.at[b], gsems[b])

    def wait_gather(g, b):
        idx_ref = idx_v.at[pl.ds(g * CHUNK_R, CHUNK_R)]
        pltpu.make_async_copy(table_hbm.at[idx_ref], rows_v.at[b],
                              gsems[b]).wait()

    def compute(g, b):
        scales = [sb_v[g * CHUNK_R + r] for r in range(CHUNK_R)]

        @plsc.parallel_loop(0, D_VECS, step=1, unroll=4)
        def _(d):
            col = pl.ds(d * LANES, LANES)
            for t in range(CHUNK_T):
                r0 = rows_v[b, 2 * t, col]
                r1 = rows_v[b, 2 * t + 1, col]
                out_v[b, t, col] = (scales[2 * t] * r0
                                    + scales[2 * t + 1] * r1)

    def start_write(g, b):
        pltpu.async_copy(out_v.at[b],
                         out_hbm.at[pl.ds(tok_base + g * CHUNK_T, CHUNK_T)],
                         wsems[b])

    def wait_write(b):
        pltpu.make_async_copy(out_v.at[b],
                              out_hbm.at[pl.ds(tok_base, CHUNK_T)],
                              wsems[b]).wait()

    start_gather(0, 0)

    def pair_body(j, carry):
        g0 = 2 * j
        start_gather(g0 + 1, 1)
        wait_gather(g0, 0)

        @pl.when(j > 0)
        def _():
            wait_write(0)

        compute(g0, 0)

        @pl.when(j < N_PAIRS - 1)
        def _():
            start_gather(g0 + 2, 0)

        start_write(g0, 0)
        wait_gather(g0 + 1, 1)

        @pl.when(j > 0)
        def _():
            wait_write(1)

        compute(g0 + 1, 1)
        start_write(g0 + 1, 1)
        return carry

    lax.fori_loop(0, N_PAIRS, pair_body, 0)
    wait_write(0)
    wait_write(1)


@jax.jit
def kernel(moe_output, scores, mapped_slots):
    idx = mapped_slots.astype(jnp.int32)
    scores_b = jnp.broadcast_to(scores[:, None], (N_SLOTS, LANES))

    mesh = plsc.VectorSubcoreMesh(core_axis_name="c", subcore_axis_name="s",
                                  num_cores=2, num_subcores=16)
    run = pl.kernel(
        _sc_body,
        out_type=jax.ShapeDtypeStruct((N_TOKENS, HIDDEN), jnp.float32),
        mesh=mesh,
        scratch_types=[
            pltpu.VMEM((SLOT_PER_WORKER,), jnp.int32),
            pltpu.VMEM((SLOT_PER_WORKER, LANES), jnp.float32),
            pltpu.VMEM((2, CHUNK_R, HIDDEN), jnp.float32),
            pltpu.VMEM((2, CHUNK_T, HIDDEN), jnp.float32),
            pltpu.SemaphoreType.DMA,
            pltpu.SemaphoreType.DMA,
            pltpu.SemaphoreType.DMA,
            pltpu.SemaphoreType.DMA,
        ],
    )
    return run(moe_output, idx, scores_b)


# speedup vs baseline: 2.4853x; 1.0891x over previous
import jax
import jax.numpy as jnp
from jax import lax
from jax.experimental import pallas as pl
from jax.experimental.pallas import tpu as pltpu
from jax.experimental.pallas import tpu_sc as plsc

TOP_K = 2
N_TOKENS = 8192
HIDDEN = 2048
N_SLOTS = N_TOKENS * TOP_K

NUM_WORKERS = 32
TOK_PER_WORKER = N_TOKENS // NUM_WORKERS
SLOT_PER_WORKER = TOP_K * TOK_PER_WORKER
CHUNK_T = 4
CHUNK_R = CHUNK_T * TOP_K
N_CHUNKS = TOK_PER_WORKER // CHUNK_T
N_PAIRS = N_CHUNKS // 2
LANES = 16
D_VECS = HIDDEN // LANES


def _sc_body(table_hbm, idx_hbm, sc_hbm, out_hbm, idx_v, sc_v, rows_v,
             out_v, gsem0, gsem1, wsem0, wsem1):
    c = lax.axis_index("c")
    s = lax.axis_index("s")
    wid = s * 2 + c
    slot_base = wid * SLOT_PER_WORKER
    tok_base = wid * TOK_PER_WORKER

    pltpu.sync_copy(idx_hbm.at[pl.ds(slot_base, SLOT_PER_WORKER)], idx_v)
    pltpu.sync_copy(sc_hbm.at[pl.ds(slot_base, SLOT_PER_WORKER)], sc_v)

    gsems = (gsem0, gsem1)
    wsems = (wsem0, wsem1)

    def start_gather(g, b):
        idx_ref = idx_v.at[pl.ds(g * CHUNK_R, CHUNK_R)]
        pltpu.async_copy(table_hbm.at[idx_ref], rows_v.at[b], gsems[b])

    def wait_gather(g, b):
        idx_ref = idx_v.at[pl.ds(g * CHUNK_R, CHUNK_R)]
        pltpu.make_async_copy(table_hbm.at[idx_ref], rows_v.at[b],
                              gsems[b]).wait()

    _bcast_dnums = lax.GatherDimensionNumbers(
        offset_dims=(), collapsed_slice_dims=(0,), start_index_map=(0,))

    def compute(g, b, sv, lane0):
        scales = [
            lax.gather(sv, jnp.full((LANES, 1), lane0 + r, jnp.int32),
                       _bcast_dnums, (1,),
                       mode=lax.GatherScatterMode.PROMISE_IN_BOUNDS)
            for r in range(CHUNK_R)
        ]

        @plsc.parallel_loop(0, D_VECS, step=1, unroll=4)
        def _(d):
            col = pl.ds(d * LANES, LANES)
            for t in range(CHUNK_T):
                r0 = rows_v[b, 2 * t, col]
                r1 = rows_v[b, 2 * t + 1, col]
                out_v[b, t, col] = (scales[2 * t] * r0
                                    + scales[2 * t + 1] * r1)

    def start_write(g, b):
        pltpu.async_copy(out_v.at[b],
                         out_hbm.at[pl.ds(tok_base + g * CHUNK_T, CHUNK_T)],
                         wsems[b])

    def wait_write(b):
        pltpu.make_async_copy(out_v.at[b],
                              out_hbm.at[pl.ds(tok_base, CHUNK_T)],
                              wsems[b]).wait()

    start_gather(0, 0)

    def pair_body(j, carry):
        g0 = 2 * j
        start_gather(g0 + 1, 1)
        wait_gather(g0, 0)

        @pl.when(j > 0)
        def _():
            wait_write(0)

        sv = sc_v[pl.ds(g0 * CHUNK_R, 2 * CHUNK_R)]
        compute(g0, 0, sv, 0)

        @pl.when(j < N_PAIRS - 1)
        def _():
            start_gather(g0 + 2, 0)

        start_write(g0, 0)
        wait_gather(g0 + 1, 1)

        @pl.when(j > 0)
        def _():
            wait_write(1)

        compute(g0 + 1, 1, sv, CHUNK_R)
        start_write(g0 + 1, 1)
        return carry

    lax.fori_loop(0, N_PAIRS, pair_body, 0)
    wait_write(0)
    wait_write(1)


@jax.jit
def kernel(moe_output, scores, mapped_slots):
    idx = mapped_slots.astype(jnp.int32)

    mesh = plsc.VectorSubcoreMesh(core_axis_name="c", subcore_axis_name="s",
                                  num_cores=2, num_subcores=16)
    run = pl.kernel(
        _sc_body,
        out_type=jax.ShapeDtypeStruct((N_TOKENS, HIDDEN), jnp.float32),
        mesh=mesh,
        scratch_types=[
            pltpu.VMEM((SLOT_PER_WORKER,), jnp.int32),
            pltpu.VMEM((SLOT_PER_WORKER,), jnp.float32),
            pltpu.VMEM((2, CHUNK_R, HIDDEN), jnp.float32),
            pltpu.VMEM((2, CHUNK_T, HIDDEN), jnp.float32),
            pltpu.SemaphoreType.DMA,
            pltpu.SemaphoreType.DMA,
            pltpu.SemaphoreType.DMA,
            pltpu.SemaphoreType.DMA,
        ],
    )
    return run(moe_output, idx, scores)


# speedup vs baseline: 2.4968x; 1.0046x over previous
import jax
import jax.numpy as jnp
from jax import lax
from jax.experimental import pallas as pl
from jax.experimental.pallas import tpu as pltpu
from jax.experimental.pallas import tpu_sc as plsc

TOP_K = 2
N_TOKENS = 8192
HIDDEN = 2048
N_SLOTS = N_TOKENS * TOP_K

NUM_WORKERS = 32
TOK_PER_WORKER = N_TOKENS // NUM_WORKERS
SLOT_PER_WORKER = TOP_K * TOK_PER_WORKER
CHUNK_T = 4
CHUNK_R = CHUNK_T * TOP_K
N_CHUNKS = TOK_PER_WORKER // CHUNK_T
N_PAIRS = N_CHUNKS // 2
LANES = 16
D_VECS = HIDDEN // LANES


def _sc_body(table_hbm, idx_hbm, sc_hbm, out_hbm, idx_v, sc_v, rows_v,
             out_v, gsem0, gsem1, wsem0, wsem1):
    c = lax.axis_index("c")
    s = lax.axis_index("s")
    wid = s * 2 + c
    slot_base = wid * SLOT_PER_WORKER
    tok_base = wid * TOK_PER_WORKER

    gsems = (gsem0, gsem1)
    wsems = (wsem0, wsem1)

    def start_gather(g, b):
        idx_ref = idx_v.at[pl.ds(g * CHUNK_R, CHUNK_R)]
        pltpu.async_copy(table_hbm.at[idx_ref], rows_v.at[b], gsems[b])

    def wait_gather(g, b):
        idx_ref = idx_v.at[pl.ds(g * CHUNK_R, CHUNK_R)]
        pltpu.make_async_copy(table_hbm.at[idx_ref], rows_v.at[b],
                              gsems[b]).wait()

    _bcast_dnums = lax.GatherDimensionNumbers(
        offset_dims=(), collapsed_slice_dims=(0,), start_index_map=(0,))

    def compute(g, b, sv, lane0):
        scales = [
            lax.gather(sv, jnp.full((LANES, 1), lane0 + r, jnp.int32),
                       _bcast_dnums, (1,),
                       mode=lax.GatherScatterMode.PROMISE_IN_BOUNDS)
            for r in range(CHUNK_R)
        ]

        @plsc.parallel_loop(0, D_VECS, step=1, unroll=8)
        def _(d):
            col = pl.ds(d * LANES, LANES)
            for t in range(CHUNK_T):
                r0 = rows_v[b, 2 * t, col]
                r1 = rows_v[b, 2 * t + 1, col]
                out_v[b, t, col] = (scales[2 * t] * r0
                                    + scales[2 * t + 1] * r1)

    def start_write(g, b):
        pltpu.async_copy(out_v.at[b],
                         out_hbm.at[pl.ds(tok_base + g * CHUNK_T, CHUNK_T)],
                         wsems[b])

    def wait_write(b):
        pltpu.make_async_copy(out_v.at[b],
                              out_hbm.at[pl.ds(tok_base, CHUNK_T)],
                              wsems[b]).wait()

    pltpu.sync_copy(idx_hbm.at[pl.ds(slot_base, SLOT_PER_WORKER)], idx_v)
    start_gather(0, 0)
    pltpu.sync_copy(sc_hbm.at[pl.ds(slot_base, SLOT_PER_WORKER)], sc_v)

    def pair_body(j, carry):
        g0 = 2 * j
        start_gather(g0 + 1, 1)
        wait_gather(g0, 0)

        @pl.when(j > 0)
        def _():
            wait_write(0)

        sv = sc_v[pl.ds(g0 * CHUNK_R, 2 * CHUNK_R)]
        compute(g0, 0, sv, 0)

        @pl.when(j < N_PAIRS - 1)
        def _():
            start_gather(g0 + 2, 0)

        start_write(g0, 0)
        wait_gather(g0 + 1, 1)

        @pl.when(j > 0)
        def _():
            wait_write(1)

        compute(g0 + 1, 1, sv, CHUNK_R)
        start_write(g0 + 1, 1)
        return carry

    lax.fori_loop(0, N_PAIRS, pair_body, 0)
    wait_write(0)
    wait_write(1)


@jax.jit
def kernel(moe_output, scores, mapped_slots):
    idx = mapped_slots.astype(jnp.int32)

    mesh = plsc.VectorSubcoreMesh(core_axis_name="c", subcore_axis_name="s",
                                  num_cores=2, num_subcores=16)
    run = pl.kernel(
        _sc_body,
        out_type=jax.ShapeDtypeStruct((N_TOKENS, HIDDEN), jnp.float32),
        mesh=mesh,
        scratch_types=[
            pltpu.VMEM((SLOT_PER_WORKER,), jnp.int32),
            pltpu.VMEM((SLOT_PER_WORKER,), jnp.float32),
            pltpu.VMEM((2, CHUNK_R, HIDDEN), jnp.float32),
            pltpu.VMEM((2, CHUNK_T, HIDDEN), jnp.float32),
            pltpu.SemaphoreType.DMA,
            pltpu.SemaphoreType.DMA,
            pltpu.SemaphoreType.DMA,
            pltpu.SemaphoreType.DMA,
        ],
    )
    return run(moe_output, idx, scores)


# speedup vs baseline: 2.5987x; 1.0408x over previous
import jax
import jax.numpy as jnp
from jax import lax
from jax.experimental import pallas as pl
from jax.experimental.pallas import tpu as pltpu
from jax.experimental.pallas import tpu_sc as plsc

TOP_K = 2
N_TOKENS = 8192
HIDDEN = 2048
N_SLOTS = N_TOKENS * TOP_K

NUM_WORKERS = 32
TOK_PER_WORKER = N_TOKENS // NUM_WORKERS
SLOT_PER_WORKER = TOP_K * TOK_PER_WORKER
CHUNK_T = 2
CHUNK_R = CHUNK_T * TOP_K
N_CHUNKS = TOK_PER_WORKER // CHUNK_T
NB = 4
DEPTH = NB - 1
N_OUTER = N_CHUNKS // NB
OUT_T = 2 * CHUNK_T
IDXPAD_PER_WORKER = N_CHUNKS * 8
LANES = 16
D_VECS = HIDDEN // LANES


def _sc_body(table_hbm, idx_hbm, sc_hbm, out_hbm, idx_v, sc_v, rows_v,
             out_v, gsem0, gsem1, gsem2, gsem3, wsem0, wsem1):
    c = lax.axis_index("c")
    s = lax.axis_index("s")
    wid = s * 2 + c
    slot_base = wid * SLOT_PER_WORKER
    tok_base = wid * TOK_PER_WORKER

    gsems = (gsem0, gsem1, gsem2, gsem3)
    wsems = (wsem0, wsem1)

    def start_gather(g, b):
        idx_ref = idx_v.at[pl.ds(g * 8, CHUNK_R)]
        pltpu.async_copy(table_hbm.at[idx_ref], rows_v.at[b], gsems[b])

    def wait_gather(g, b):
        idx_ref = idx_v.at[pl.ds(g * 8, CHUNK_R)]
        pltpu.make_async_copy(table_hbm.at[idx_ref], rows_v.at[b],
                              gsems[b]).wait()

    def start_write(p, o):
        pltpu.async_copy(out_v.at[o],
                         out_hbm.at[pl.ds(tok_base + p * OUT_T, OUT_T)],
                         wsems[o])

    def wait_write(o):
        pltpu.make_async_copy(out_v.at[o],
                              out_hbm.at[pl.ds(tok_base, OUT_T)],
                              wsems[o]).wait()

    _bcast_dnums = lax.GatherDimensionNumbers(
        offset_dims=(), collapsed_slice_dims=(0,), start_index_map=(0,))

    def lane_bcast(sv, lane):
        return lax.gather(sv, jnp.full((LANES, 1), lane, jnp.int32),
                          _bcast_dnums, (1,),
                          mode=lax.GatherScatterMode.PROMISE_IN_BOUNDS)

    pltpu.sync_copy(
        idx_hbm.at[pl.ds(wid * IDXPAD_PER_WORKER, IDXPAD_PER_WORKER)], idx_v)
    for j in range(DEPTH):
        start_gather(j, j)
    pltpu.sync_copy(sc_hbm.at[pl.ds(slot_base, SLOT_PER_WORKER)], sc_v)

    def outer_body(k, carry):
        g0 = NB * k
        sv = sc_v[pl.ds(k * LANES, LANES)]
        scales = [lane_bcast(sv, r) for r in range(LANES)]

        for j in range(NB):
            g = g0 + j
            b = j
            o = j // 2
            h = j % 2
            wait_gather(g, b)

            @pl.when(g + DEPTH < N_CHUNKS)
            def _(g=g, b=(j + DEPTH) % NB):
                start_gather(g + DEPTH, b)

            if h == 0:
                @pl.when(k > 0)
                def _(o=o):
                    wait_write(o)

            @plsc.parallel_loop(0, D_VECS, step=1, unroll=8)
            def _(d, b=b, o=o, h=h, j=j):
                col = pl.ds(d * LANES, LANES)
                for t in range(CHUNK_T):
                    r0 = rows_v[b, 2 * t, col]
                    r1 = rows_v[b, 2 * t + 1, col]
                    out_v[o, h * CHUNK_T + t, col] = (
                        scales[4 * j + 2 * t] * r0
                        + scales[4 * j + 2 * t + 1] * r1)

            if h == 1:
                start_write(2 * k + o, o)
        return carry

    lax.fori_loop(0, N_OUTER, outer_body, 0)
    wait_write(0)
    wait_write(1)


@jax.jit
def kernel(moe_output, scores, mapped_slots):
    idx = mapped_slots.astype(jnp.int32)
    idx2 = idx.reshape(-1, CHUNK_R)
    idx_pad = jnp.concatenate([idx2, idx2], axis=1).reshape(-1)

    mesh = plsc.VectorSubcoreMesh(core_axis_name="c", subcore_axis_name="s",
                                  num_cores=2, num_subcores=16)
    run = pl.kernel(
        _sc_body,
        out_type=jax.ShapeDtypeStruct((N_TOKENS, HIDDEN), jnp.float32),
        mesh=mesh,
        scratch_types=[
            pltpu.VMEM((IDXPAD_PER_WORKER,), jnp.int32),
            pltpu.VMEM((SLOT_PER_WORKER,), jnp.float32),
            pltpu.VMEM((NB, CHUNK_R, HIDDEN), jnp.float32),
            pltpu.VMEM((2, OUT_T, HIDDEN), jnp.float32),
            pltpu.SemaphoreType.DMA,
            pltpu.SemaphoreType.DMA,
            pltpu.SemaphoreType.DMA,
            pltpu.SemaphoreType.DMA,
            pltpu.SemaphoreType.DMA,
            pltpu.SemaphoreType.DMA,
        ],
    )
    return run(moe_output, idx_pad, scores)


# speedup vs baseline: 2.6004x; 1.0006x over previous
import jax
import jax.numpy as jnp
from jax import lax
from jax.experimental import pallas as pl
from jax.experimental.pallas import tpu as pltpu
from jax.experimental.pallas import tpu_sc as plsc

TOP_K = 2
N_TOKENS = 8192
HIDDEN = 2048
N_SLOTS = N_TOKENS * TOP_K

NUM_WORKERS = 32
TOK_PER_WORKER = N_TOKENS // NUM_WORKERS
SLOT_PER_WORKER = TOP_K * TOK_PER_WORKER
CHUNK_T = 2
CHUNK_R = CHUNK_T * TOP_K
N_CHUNKS = TOK_PER_WORKER // CHUNK_T
NB = 4
DEPTH = NB - 1
N_OUTER = N_CHUNKS // NB
OUT_T = 2 * CHUNK_T
IDXPAD_PER_WORKER = N_CHUNKS * 8
LANES = 16
D_VECS = HIDDEN // LANES


def _sc_body(table_hbm, idx_hbm, sc_hbm, out_hbm, idx_v, sc_v, rows_v,
             out_v, gsem0, gsem1, gsem2, gsem3, wsem0, wsem1, wsem2):
    c = lax.axis_index("c")
    s = lax.axis_index("s")
    wid = s * 2 + c
    slot_base = wid * SLOT_PER_WORKER
    tok_base = wid * TOK_PER_WORKER

    gsems = (gsem0, gsem1, gsem2, gsem3)
    wsems = (wsem0, wsem1, wsem2)

    def start_gather(g, b):
        idx_ref = idx_v.at[pl.ds(g * 8, CHUNK_R)]
        pltpu.async_copy(table_hbm.at[idx_ref], rows_v.at[b], gsems[b])

    def wait_gather(g, b):
        idx_ref = idx_v.at[pl.ds(g * 8, CHUNK_R)]
        pltpu.make_async_copy(table_hbm.at[idx_ref], rows_v.at[b],
                              gsems[b]).wait()

    def start_write(p, o):
        for o_s in range(3):
            @pl.when(o == o_s)
            def _(o_s=o_s):
                pltpu.async_copy(
                    out_v.at[o_s],
                    out_hbm.at[pl.ds(tok_base + p * OUT_T, OUT_T)],
                    wsems[o_s])

    def wait_write_static(o_s):
        pltpu.make_async_copy(out_v.at[o_s],
                              out_hbm.at[pl.ds(tok_base, OUT_T)],
                              wsems[o_s]).wait()

    def wait_write(o):
        for o_s in range(3):
            @pl.when(o == o_s)
            def _(o_s=o_s):
                wait_write_static(o_s)

    _bcast_dnums = lax.GatherDimensionNumbers(
        offset_dims=(), collapsed_slice_dims=(0,), start_index_map=(0,))

    def lane_bcast(sv, lane):
        return lax.gather(sv, jnp.full((LANES, 1), lane, jnp.int32),
                          _bcast_dnums, (1,),
                          mode=lax.GatherScatterMode.PROMISE_IN_BOUNDS)

    pltpu.sync_copy(
        idx_hbm.at[pl.ds(wid * IDXPAD_PER_WORKER, IDXPAD_PER_WORKER)], idx_v)
    for j in range(DEPTH):
        start_gather(j, j)
    pltpu.sync_copy(sc_hbm.at[pl.ds(slot_base, SLOT_PER_WORKER)], sc_v)

    def outer_body(k, carry):
        g0 = NB * k
        sv = sc_v[pl.ds(k * LANES, LANES)]
        scales = [lane_bcast(sv, r) for r in range(LANES)]

        for j in range(NB):
            g = g0 + j
            b = j
            p = 2 * k + j // 2
            o = lax.rem(p, 3)
            h = j % 2
            wait_gather(g, b)

            @pl.when(g + DEPTH < N_CHUNKS)
            def _(g=g, b=(j + DEPTH) % NB):
                start_gather(g + DEPTH, b)

            if h == 0:
                @pl.when(p >= 3)
                def _(o=o):
                    wait_write(o)

            @plsc.parallel_loop(0, D_VECS, step=1, unroll=8)
            def _(d, b=b, o=o, h=h, j=j):
                col = pl.ds(d * LANES, LANES)
                for t in range(CHUNK_T):
                    r0 = rows_v[b, 2 * t, col]
                    r1 = rows_v[b, 2 * t + 1, col]
                    out_v[o, h * CHUNK_T + t, col] = (
                        scales[4 * j + 2 * t] * r0
                        + scales[4 * j + 2 * t + 1] * r1)

            if h == 1:
                start_write(p, o)
        return carry

    lax.fori_loop(0, N_OUTER, outer_body, 0)
    wait_write_static(0)
    wait_write_static(1)
    wait_write_static(2)


@jax.jit
def kernel(moe_output, scores, mapped_slots):
    idx = mapped_slots.astype(jnp.int32)
    idx2 = idx.reshape(-1, CHUNK_R)
    idx_pad = jnp.concatenate([idx2, idx2], axis=1).reshape(-1)

    mesh = plsc.VectorSubcoreMesh(core_axis_name="c", subcore_axis_name="s",
                                  num_cores=2, num_subcores=16)
    run = pl.kernel(
        _sc_body,
        out_type=jax.ShapeDtypeStruct((N_TOKENS, HIDDEN), jnp.float32),
        mesh=mesh,
        scratch_types=[
            pltpu.VMEM((IDXPAD_PER_WORKER,), jnp.int32),
            pltpu.VMEM((SLOT_PER_WORKER,), jnp.float32),
            pltpu.VMEM((NB, CHUNK_R, HIDDEN), jnp.float32),
            pltpu.VMEM((3, OUT_T, HIDDEN), jnp.float32),
            pltpu.SemaphoreType.DMA,
            pltpu.SemaphoreType.DMA,
            pltpu.SemaphoreType.DMA,
            pltpu.SemaphoreType.DMA,
            pltpu.SemaphoreType.DMA,
            pltpu.SemaphoreType.DMA,
            pltpu.SemaphoreType.DMA,
        ],
    )
    return run(moe_output, idx_pad, scores)
